# baseline (speedup 1.0000x reference)
import functools

import jax
import jax.numpy as jnp
from jax import lax
from jax.experimental import pallas as pl
from jax.experimental.pallas import tpu as pltpu
from jax.experimental.pallas import tpu_sc as plsc

NUM_EMBEDDINGS = 1000000
EMBEDDING_DIM = 32
BATCH = 16384
FIELDS = 26

_TOTAL = BATCH * FIELDS
_NW = 32
_PER_W = _TOTAL // _NW
_CHUNK = 1664
_NCHUNK = _PER_W // _CHUNK
_NBUF = 2

_FULL_COLS = NUM_EMBEDDINGS // 128
_REM = NUM_EMBEDDINGS - _FULL_COLS * 128
_COLS_PER_W = _FULL_COLS // _NW
_COLS_EXTRA = _FULL_COLS - _COLS_PER_W * _NW

assert _PER_W % _CHUNK == 0 and _CHUNK % 8 == 0


def _worker_id():
    return lax.axis_index("s") * 2 + lax.axis_index("c")


def _make_detranspose():
    mesh = plsc.VectorSubcoreMesh(core_axis_name="c", subcore_axis_name="s")

    @functools.partial(
        pl.kernel,
        mesh=mesh,
        out_type=jax.ShapeDtypeStruct((NUM_EMBEDDINGS * EMBEDDING_DIM,),
                                      jnp.float32),
        scratch_types=[
            pltpu.VMEM((32, 128), jnp.float32),
            pltpu.VMEM((4096,), jnp.float32),
            pltpu.VMEM((64,), jnp.float32),
        ],
        compiler_params=pltpu.CompilerParams(use_tc_tiling_on_sc=True,
                                             needs_layout_passes=False),
    )
    def detr_kernel(wt_hbm, flat_hbm, inbuf, stage, remline):
        wid = _worker_id()
        n_cols = jnp.where(wid < _COLS_EXTRA, _COLS_PER_W + 1, _COLS_PER_W)
        col0 = wid * _COLS_PER_W + jnp.minimum(wid, _COLS_EXTRA)
        lane = lax.iota(jnp.int32, 16)

        def body(k, _):
            col = col0 + k
            pltpu.sync_copy(wt_hbm.at[:, pl.ds(col * 128, 128)], inbuf)
            for j in range(32):
                for l in range(8):
                    v = inbuf[j, pl.ds(l * 16, 16)]
                    plsc.store_scatter(
                        stage, [(l * 16 + lane) * 32 + j], v)
            pltpu.sync_copy(stage, flat_hbm.at[pl.ds(col * 4096, 4096)])
            return 0

        lax.fori_loop(0, n_cols, body, 0)

        @pl.when(wid == _NW - 1)
        def _():
            for j in range(32):
                pltpu.sync_copy(wt_hbm.at[j, pl.ds(_FULL_COLS * 128, 64)],
                                remline)
                for l in range(4):
                    v = remline[pl.ds(l * 16, 16)]
                    plsc.store_scatter(
                        stage, [(l * 16 + lane) * 32 + j], v)
            pltpu.sync_copy(stage.at[pl.ds(0, 64 * 32)],
                            flat_hbm.at[pl.ds(_FULL_COLS * 4096, 64 * 32)])

    return detr_kernel


def _make_gather():
    mesh = plsc.VectorSubcoreMesh(core_axis_name="c", subcore_axis_name="s")

    @functools.partial(
        pl.kernel,
        mesh=mesh,
        out_type=jax.ShapeDtypeStruct((_TOTAL, EMBEDDING_DIM), jnp.float32),
        scratch_types=[
            pltpu.VMEM((_PER_W,), jnp.int32),
            pltpu.VMEM((_NBUF, _CHUNK, EMBEDDING_DIM), jnp.float32),
            [pltpu.SemaphoreType.DMA] * _NBUF,
            [pltpu.SemaphoreType.DMA] * _NBUF,
        ],
        compiler_params=pltpu.CompilerParams(use_tc_tiling_on_sc=False),
    )
    def emb_kernel(table_hbm, idx_hbm, out_hbm, idx_v, rows_v, gsems, ssems):
        wid = _worker_id()
        base = wid * _PER_W
        pltpu.sync_copy(idx_hbm.at[pl.ds(base, _PER_W)], idx_v)

        def start_gather(g, b):
            return pltpu.async_copy(
                table_hbm.at[idx_v.at[pl.ds(g * _CHUNK, _CHUNK)]],
                rows_v.at[b],
                gsems[b],
            )

        def start_store(g, b):
            return pltpu.async_copy(
                rows_v.at[b],
                out_hbm.at[pl.ds(base + g * _CHUNK, _CHUNK)],
                ssems[b],
            )

        gathers = {}
        stores = {}
        gathers[0] = start_gather(0, 0)
        for g in range(_NCHUNK):
            b = g % _NBUF
            nb = (g + 1) % _NBUF
            if g + 1 < _NCHUNK:
                prev = g + 1 - _NBUF
                if prev >= 0:
                    stores.pop(prev).wait()
                gathers[g + 1] = start_gather(g + 1, nb)
            gathers.pop(g).wait()
            stores[g] = start_store(g, b)
        for g in sorted(stores):
            stores.pop(g).wait()

    return emb_kernel


_DETR = _make_detranspose()
_EMB = _make_gather()


@jax.jit
def kernel(x, weight):
    idx = x.reshape(-1).astype(jnp.int32)
    flat_table = _DETR(weight.T)
    table = flat_table.reshape(NUM_EMBEDDINGS, EMBEDDING_DIM)
    out = _EMB(table, idx)
    return out.reshape(BATCH, FIELDS, EMBEDDING_DIM)


# speedup vs baseline: 2.1319x; 2.1319x over previous
import functools

import jax
import jax.numpy as jnp
from jax import lax
from jax.experimental import pallas as pl
from jax.experimental.pallas import tpu as pltpu
from jax.experimental.pallas import tpu_sc as plsc

NUM_EMBEDDINGS = 1000000
EMBEDDING_DIM = 32
BATCH = 16384
FIELDS = 26

_TOTAL = BATCH * FIELDS
_NW = 32
_PER_W = _TOTAL // _NW
_CHUNK = 1664
_NCHUNK = _PER_W // _CHUNK
_NBUF = 2

_FULL_COLS = NUM_EMBEDDINGS // 128
_REM = NUM_EMBEDDINGS - _FULL_COLS * 128
_COLS_PER_W = _FULL_COLS // _NW
_COLS_EXTRA = _FULL_COLS - _COLS_PER_W * _NW

assert _PER_W % _CHUNK == 0 and _CHUNK % 8 == 0


def _worker_id():
    return lax.axis_index("s") * 2 + lax.axis_index("c")


def _make_detranspose():
    mesh = plsc.VectorSubcoreMesh(core_axis_name="c", subcore_axis_name="s")

    @functools.partial(
        pl.kernel,
        mesh=mesh,
        out_type=jax.ShapeDtypeStruct((NUM_EMBEDDINGS * EMBEDDING_DIM // 128,
                                       128), jnp.float32),
        scratch_types=[
            pltpu.VMEM((2, 32, 128), jnp.float32),
            pltpu.VMEM((2, 32, 128), jnp.float32),
            pltpu.VMEM((16, 128), jnp.float32),
            [pltpu.SemaphoreType.DMA] * 2,
            [pltpu.SemaphoreType.DMA] * 2,
        ],
        compiler_params=pltpu.CompilerParams(use_tc_tiling_on_sc=True,
                                             needs_layout_passes=False,
                                             disable_bounds_checks=True),
    )
    def detr_kernel(wt_hbm, wtail_hbm, flat_hbm, inbuf, stage, rembuf,
                    isems, osems):
        wid = _worker_id()
        n_cols = jnp.where(wid < _COLS_EXTRA, _COLS_PER_W + 1, _COLS_PER_W)
        col0 = wid * _COLS_PER_W + jnp.minimum(wid, _COLS_EXTRA)
        end = col0 + n_cols
        lane = lax.iota(jnp.int32, 16)
        jmod = [(lane + d) % 16 for d in range(16)]
        jconst = [[jmod[d] + jb * 16 for jb in range(2)] for d in range(16)]
        ivec = [lane + l * 16 for l in range(8)]
        dbase = [[lane * 32 + jmod[d] + jb * 16 for jb in range(2)]
                 for d in range(16)]
        scol = [[lax.bitwise_and(dbase[d][jb], 127) for jb in range(2)]
                for d in range(16)]
        srow0 = [[lax.shift_right_logical(dbase[d][jb], 7) for jb in range(2)]
                 for d in range(16)]

        def start_in(col, b):
            return pltpu.async_copy(
                wt_hbm.at[:, pl.ds(col * 128, 128)], inbuf.at[b], isems[b])

        def wait_in(col, b):
            pltpu.make_async_copy(
                wt_hbm.at[:, pl.ds(col * 128, 128)], inbuf.at[b],
                isems[b]).wait()

        def start_out(col, b):
            return pltpu.async_copy(
                stage.at[b], flat_hbm.at[pl.ds(col * 32, 32), :], osems[b])

        def wait_out(col, b):
            pltpu.make_async_copy(
                stage.at[b], flat_hbm.at[pl.ds(col * 32, 32), :],
                osems[b]).wait()

        start_in(col0, 0)

        def body(kk, _):
            for i in (0, 1):
                col = col0 + kk * 2 + i
                b = i

                @pl.when(col < end)
                def _():
                    @pl.when(col + 1 < end)
                    def _():
                        start_in(col + 1, 1 - b)

                    wait_in(col, b)

                    @pl.when(col - 2 >= col0)
                    def _():
                        wait_out(col - 2, b)

                    def lbody(l, _):
                        iv = lane + l * 16
                        for jb in range(2):
                            for d in range(16):
                                v = plsc.load_gather(
                                    inbuf.at[b],
                                    [jconst[d][jb], iv])
                                plsc.store_scatter(
                                    stage.at[b],
                                    [srow0[d][jb] + l * 4, scol[d][jb]], v)
                        return 0

                    lax.fori_loop(0, 8, lbody, 0)
                    start_out(col, b)
            return 0

        lax.fori_loop(0, (_COLS_PER_W + 2) // 2, body, 0)

        @pl.when(n_cols % 2 == 0)
        def _():
            wait_out(end - 2, 0)
            wait_out(end - 1, 1)

        @pl.when(n_cols % 2 == 1)
        def _():
            wait_out(end - 2, 1)
            wait_out(end - 1, 0)

        @pl.when(wid == _NW - 1)
        def _():
            pltpu.sync_copy(wtail_hbm, rembuf)
            pltpu.sync_copy(rembuf,
                            flat_hbm.at[pl.ds(_FULL_COLS * 32, 16), :])

    return detr_kernel


def _make_gather():
    mesh = plsc.VectorSubcoreMesh(core_axis_name="c", subcore_axis_name="s")

    @functools.partial(
        pl.kernel,
        mesh=mesh,
        out_type=jax.ShapeDtypeStruct((_TOTAL, EMBEDDING_DIM), jnp.float32),
        scratch_types=[
            pltpu.VMEM((_PER_W,), jnp.int32),
            pltpu.VMEM((_NBUF, _CHUNK, EMBEDDING_DIM), jnp.float32),
            [pltpu.SemaphoreType.DMA] * _NBUF,
            [pltpu.SemaphoreType.DMA] * _NBUF,
        ],
        compiler_params=pltpu.CompilerParams(use_tc_tiling_on_sc=False),
    )
    def emb_kernel(table_hbm, idx_hbm, out_hbm, idx_v, rows_v, gsems, ssems):
        wid = _worker_id()
        base = wid * _PER_W
        pltpu.sync_copy(idx_hbm.at[pl.ds(base, _PER_W)], idx_v)

        def start_gather(g, b):
            return pltpu.async_copy(
                table_hbm.at[idx_v.at[pl.ds(g * _CHUNK, _CHUNK)]],
                rows_v.at[b],
                gsems[b],
            )

        def start_store(g, b):
            return pltpu.async_copy(
                rows_v.at[b],
                out_hbm.at[pl.ds(base + g * _CHUNK, _CHUNK)],
                ssems[b],
            )

        gathers = {}
        stores = {}
        gathers[0] = start_gather(0, 0)
        for g in range(_NCHUNK):
            b = g % _NBUF
            nb = (g + 1) % _NBUF
            if g + 1 < _NCHUNK:
                prev = g + 1 - _NBUF
                if prev >= 0:
                    stores.pop(prev).wait()
                gathers[g + 1] = start_gather(g + 1, nb)
            gathers.pop(g).wait()
            stores[g] = start_store(g, b)
        for g in sorted(stores):
            stores.pop(g).wait()

    return emb_kernel


_DETR = _make_detranspose()
_EMB = _make_gather()


@jax.jit
def kernel(x, weight):
    idx = x.reshape(-1).astype(jnp.int32)
    wtail = weight[_FULL_COLS * 128:].reshape(16, 128)
    flat_table = _DETR(weight.T, wtail)
    table = flat_table.reshape(NUM_EMBEDDINGS, EMBEDDING_DIM)
    out = _EMB(table, idx)
    return out.reshape(BATCH, FIELDS, EMBEDDING_DIM)
